# baseline (speedup 1.0000x reference)
import functools

import jax
import jax.numpy as jnp
from jax.experimental import pallas as pl
from jax.experimental.pallas import tpu as pltpu

PC_RANGE = (0.0, -40.0, -3.0, 70.4, 40.0, 1.0)
PILLAR = 0.1
H = 800
W_GRID = 704
C_OUT = 32

_BLK = 2048


def _pfe_body(pts_ref, bo_ref, wm_ref, bb_ref, h_ref, seg_ref):
    pts = pts_ref[...]
    x = pts[:, 0]
    y = pts[:, 1]
    cx = jnp.floor((x - PC_RANGE[0]) / PILLAR).astype(jnp.int32)
    cy = jnp.floor((y - PC_RANGE[1]) / PILLAR).astype(jnp.int32)
    mask = (cx >= 0) & (cx < W_GRID) & (cy >= 0) & (cy < H)
    cxc = jnp.clip(cx, 0, W_GRID - 1)
    cyc = jnp.clip(cy, 0, H - 1)
    center_x = (cxc.astype(jnp.float32) + 0.5) * PILLAR + PC_RANGE[0]
    center_y = (cyc.astype(jnp.float32) + 0.5) * PILLAR + PC_RANGE[1]
    feats = jnp.concatenate(
        [pts[:, :5], (x - center_x)[:, None], (y - center_y)[:, None]], axis=1
    )
    h = jnp.dot(feats, wm_ref[...], preferred_element_type=jnp.float32) + bb_ref[...]
    h_ref[...] = jnp.maximum(h, 0.0)
    seg = bo_ref[...] + cyc * W_GRID + cxc
    seg_ref[...] = jnp.where(mask, seg, jnp.int32(H * W_GRID * 2))


def _point_stage(pts_flat, bo, wm, bb, n_pad):
    grid = n_pad // _BLK
    return pl.pallas_call(
        _pfe_body,
        grid=(grid,),
        in_specs=[
            pl.BlockSpec((_BLK, 8), lambda i: (i, 0)),
            pl.BlockSpec((_BLK,), lambda i: (i,)),
            pl.BlockSpec((7, C_OUT), lambda i: (0, 0)),
            pl.BlockSpec((1, C_OUT), lambda i: (0, 0)),
        ],
        out_specs=[
            pl.BlockSpec((_BLK, C_OUT), lambda i: (i, 0)),
            pl.BlockSpec((_BLK,), lambda i: (i,)),
        ],
        out_shape=[
            jax.ShapeDtypeStruct((n_pad, C_OUT), jnp.float32),
            jax.ShapeDtypeStruct((n_pad,), jnp.int32),
        ],
    )(pts_flat, bo, wm, bb)


def kernel(points, Wm, b, gamma, beta, mean, var):
    B, N, C = points.shape
    n_tot = B * N
    n_pad = ((n_tot + _BLK - 1) // _BLK) * _BLK
    scale = gamma / jnp.sqrt(var + 1e-5)
    wm = Wm * scale[None, :]
    bb = ((b - mean) * scale + beta)[None, :]
    pts_flat = points.reshape(n_tot, C)
    pts_flat = jnp.pad(pts_flat, ((0, n_pad - n_tot), (0, 8 - C)),
                       constant_values=-1e9)
    bo = jnp.where(jnp.arange(n_pad, dtype=jnp.int32) >= N,
                   jnp.int32(H * W_GRID), jnp.int32(0))
    h, seg = _point_stage(pts_flat, bo, wm, bb, n_pad)
    pooled = jax.ops.segment_max(h, seg, num_segments=2 * H * W_GRID + 1)
    pooled = pooled[: 2 * H * W_GRID]
    pooled = jnp.where(jnp.isfinite(pooled), pooled, 0.0)
    return pooled.reshape(B, H, W_GRID, C_OUT).transpose(0, 3, 1, 2)
